# Initial kernel scaffold; baseline (speedup 1.0000x reference)
import jax
import jax.numpy as jnp
from jax.experimental import pallas as pl


def kernel(points, q, t, sample_idx, group_idx, W1, W2, W3, W4):
    raise NotImplementedError("write your pallas kernel here")


# speedup vs baseline: 44.7960x; 44.7960x over previous
import functools

import jax
import jax.numpy as jnp
from jax import lax
from jax.experimental import pallas as pl
from jax.experimental.pallas import tpu as pltpu
from jax.experimental.pallas import tpu_sc as plsc

NC = 2
NS = 16
L = 16
NW = NC * NS


def _sc_call(pts_T, gi_w, si_w, wpack, qpack):
    B = pts_T.shape[0]
    C = 3
    N = pts_T.shape[1] // C
    PW = si_w.shape[1]
    S = gi_w.shape[1] // PW
    WPB = NW // B
    NBLK = PW // L
    NWV = wpack.shape[0] // L

    mesh = plsc.VectorSubcoreMesh(
        core_axis_name="c", subcore_axis_name="s",
        num_cores=NC, num_subcores=NS)

    @functools.partial(
        pl.kernel,
        out_type=jax.ShapeDtypeStruct((NW, C * PW), jnp.float32),
        mesh=mesh,
        scratch_types=[
            pltpu.VMEM((C * N,), jnp.float32),
            pltpu.VMEM((S * PW,), jnp.int32),
            pltpu.VMEM((PW,), jnp.int32),
            pltpu.VMEM((wpack.shape[0],), jnp.float32),
            pltpu.VMEM((B * L,), jnp.float32),
            pltpu.VMEM((C * PW,), jnp.float32),
        ],
        compiler_params=pltpu.CompilerParams(needs_layout_passes=False),
    )
    def k(pts_hbm, gi_hbm, si_hbm, wpack_hbm, qpack_hbm, out_hbm,
          pts_v, gi_v, si_v, wpack_v, qpack_v, out_v):
        wid = lax.axis_index("s") * NC + lax.axis_index("c")
        b = wid // WPB
        pltpu.sync_copy(pts_hbm.at[b], pts_v)
        pltpu.sync_copy(gi_hbm.at[wid], gi_v)
        pltpu.sync_copy(si_hbm.at[wid], si_v)
        pltpu.sync_copy(wpack_hbm, wpack_v)
        pltpu.sync_copy(qpack_hbm, qpack_v)

        wvec = [wpack_v[pl.ds(i * L, L)] for i in range(NWV)]

        def wsc(k):
            return wvec[k // L][k % L]

        w1 = [[wsc(i * 8 + j) for j in range(8)] for i in range(3)]
        w2 = [[wsc(24 + i * 8 + j) for j in range(8)] for i in range(8)]
        w3 = [[wsc(88 + i * 16 + j) for j in range(16)] for i in range(8)]
        w4 = [[wsc(216 + i * 3 + j) for j in range(3)] for i in range(16)]
        qrow = qpack_v[pl.ds(pl.multiple_of(b * L, L), L)]
        qa = [qrow[i] for i in range(4)]
        qb = [qrow[4 + i] for i in range(4)]
        tt = [qrow[8 + i] for i in range(3)]

        off_n = jnp.full((L,), N, jnp.int32)

        def blk_body(blk, carry):
            pblk = pl.multiple_of(blk * L, L)
            cidx = si_v[pl.ds(pblk, L)]
            cidx_y = cidx + off_n
            cidx_z = cidx_y + off_n
            cx = plsc.load_gather(pts_v, [cidx])
            cy = plsc.load_gather(pts_v, [cidx_y])
            cz = plsc.load_gather(pts_v, [cidx_z])

            def s_body(s, acc):
                nidx = gi_v[pl.ds(pl.multiple_of(s * PW + pblk, L), L)]
                nidx_y = nidx + off_n
                nidx_z = nidx_y + off_n
                rx = plsc.load_gather(pts_v, [nidx]) - cx
                ry = plsc.load_gather(pts_v, [nidx_y]) - cy
                rz = plsc.load_gather(pts_v, [nidx_z]) - cz
                h1 = [jnp.maximum(rx * w1[0][j] + ry * w1[1][j]
                                  + rz * w1[2][j], 0.0)
                      for j in range(8)]
                h2 = []
                for j in range(8):
                    v = h1[0] * w2[0][j]
                    for i in range(1, 8):
                        v = v + h1[i] * w2[i][j]
                    h2.append(jnp.maximum(v, 0.0))
                out = []
                for j in range(16):
                    v = h2[0] * w3[0][j]
                    for i in range(1, 8):
                        v = v + h2[i] * w3[i][j]
                    out.append(jnp.maximum(acc[j], v))
                return tuple(out)

            acc0 = tuple(jnp.zeros((L,), jnp.float32) for _ in range(16))
            feats = lax.fori_loop(0, S, s_body, acc0)

            flow = []
            for d in range(3):
                v = feats[0] * w4[0][d]
                for i in range(1, 16):
                    v = v + feats[i] * w4[i][d]
                flow.append(v)

            r0 = -(qa[1] * cx + qa[2] * cy + qa[3] * cz)
            r1 = qa[0] * cx - qa[2] * cz - qa[3] * cy
            r2 = qa[0] * cy - qa[1] * cz - qa[3] * cx
            r3 = qa[0] * cz - qa[1] * cy - qa[2] * cx
            wx = r0 * qb[1] - r1 * qb[0] - r2 * qb[3] - r3 * qb[2]
            wy = r0 * qb[2] - r1 * qb[3] - r2 * qb[0] - r3 * qb[1]
            wz = r0 * qb[3] - r1 * qb[2] - r2 * qb[1] - r3 * qb[0]

            out_v[pl.ds(pblk, L)] = wx + tt[0] + flow[0]
            out_v[pl.ds(pblk + PW, L)] = wy + tt[1] + flow[1]
            out_v[pl.ds(pblk + 2 * PW, L)] = wz + tt[2] + flow[2]
            return carry

        lax.fori_loop(0, NBLK, blk_body, 0)
        pltpu.sync_copy(out_v, out_hbm.at[wid])

    return k(pts_T, gi_w, si_w, wpack, qpack)


def kernel(points, q, t, sample_idx, group_idx, W1, W2, W3, W4):
    B, N, _ = points.shape
    P = sample_idx.shape[1]
    S = group_idx.shape[2]
    WPB = NW // B
    PW = P // WPB

    qf = jnp.reshape(q, (B, 4)).astype(jnp.float32)
    qn = qf / (jnp.sqrt(jnp.sum(qf * qf, axis=-1, keepdims=True) + 1e-10)
               + 1e-10)
    q2 = jnp.sum(qn * qn, axis=-1, keepdims=True) + 1e-10
    qinv = jnp.concatenate([qn[:, 0:1], -qn[:, 1:4]], axis=-1) / q2

    pts_T = jnp.transpose(points.astype(jnp.float32),
                          (0, 2, 1)).reshape(B, 3 * N)
    gi = group_idx.astype(jnp.int32)
    gi_w = (jnp.transpose(gi, (0, 2, 1))
            .reshape(B, S, WPB, PW)
            .transpose(0, 2, 1, 3)
            .reshape(NW, S * PW))
    si_w = sample_idx.astype(jnp.int32).reshape(NW, PW)

    wpack = jnp.concatenate([
        W1.astype(jnp.float32).ravel(), W2.astype(jnp.float32).ravel(),
        W3.astype(jnp.float32).ravel(), W4.astype(jnp.float32).ravel(),
        jnp.zeros((8,), jnp.float32)])
    qpack = jnp.concatenate([
        qn, qinv, t.astype(jnp.float32),
        jnp.zeros((B, L - 11), jnp.float32)], axis=1).ravel()

    out = _sc_call(pts_T, gi_w, si_w, wpack, qpack)
    return (out.reshape(B, WPB, 3, PW)
            .transpose(0, 1, 3, 2)
            .reshape(B, P, 3))
